# baseline (speedup 1.0000x reference)
import jax
import jax.numpy as jnp
from jax import lax
from jax.experimental import pallas as pl
from jax.experimental.pallas import tpu as pltpu
from jax.experimental.pallas import tpu_sc as plsc

NUM_CORES = 2
NUM_SUBCORES = 16
NUM_WORKERS = NUM_CORES * NUM_SUBCORES

FMT_CHUNK = 200

NB = 4


def _make_mesh():
    return plsc.VectorSubcoreMesh(
        core_axis_name="c", subcore_axis_name="s",
        num_cores=NUM_CORES, num_subcores=NUM_SUBCORES)


def _format_table(table, V, D):
    n_chunks = V // FMT_CHUNK
    assert V % FMT_CHUNK == 0 and FMT_CHUNK % 8 == 0
    n_iter = -(-n_chunks // NUM_WORKERS)
    n_iter += n_iter % 2

    @pl.kernel(
        mesh=_make_mesh(),
        compiler_params=pltpu.CompilerParams(use_tc_tiling_on_sc=True),
        out_type=jax.ShapeDtypeStruct((V, 2 * D), jnp.float32),
        scratch_types=[
            pltpu.VMEM((FMT_CHUNK, D), jnp.float32),
            pltpu.VMEM((FMT_CHUNK, D), jnp.float32),
            pltpu.VMEM((FMT_CHUNK, 2 * D), jnp.float32),
            pltpu.VMEM((FMT_CHUNK, 2 * D), jnp.float32),
            pltpu.SemaphoreType.DMA,
            pltpu.SemaphoreType.DMA,
        ],
    )
    def k1(table_hbm, fmt_hbm, n0, n1, w0, w1, sin, sout):
        wid = lax.axis_index("s") * NUM_CORES + lax.axis_index("c")

        def cid(j):
            return j * NUM_WORKERS + wid

        def fire_in(c, nbuf):
            @pl.when(c < n_chunks)
            def _():
                pltpu.async_copy(
                    table_hbm.at[pl.ds(c * FMT_CHUNK, FMT_CHUNK)], nbuf, sin)

        def wait_in(c, nbuf):
            @pl.when(c < n_chunks)
            def _():
                pltpu.make_async_copy(
                    table_hbm.at[pl.ds(c * FMT_CHUNK, FMT_CHUNK)], nbuf,
                    sin).wait()

        def widen(c, nbuf, wbuf):
            @pl.when(c < n_chunks)
            def _():
                def row(r, carry):
                    for j in range(D // 16):
                        wbuf[r, pl.ds(16 * j, 16)] = nbuf[r, pl.ds(16 * j, 16)]
                    return carry
                lax.fori_loop(0, FMT_CHUNK, row, 0)

        def fire_out(c, wbuf):
            @pl.when(c < n_chunks)
            def _():
                pltpu.async_copy(
                    wbuf, fmt_hbm.at[pl.ds(c * FMT_CHUNK, FMT_CHUNK)], sout)

        def wait_out(c, wbuf):
            @pl.when(jnp.logical_and(c >= 0, c < n_chunks))
            def _():
                pltpu.make_async_copy(
                    wbuf, fmt_hbm.at[pl.ds(c * FMT_CHUNK, FMT_CHUNK)],
                    sout).wait()

        fire_in(cid(0), n0)

        def body(i, carry):
            ca = cid(2 * i)
            cb = cid(2 * i + 1)
            fire_in(cb, n1)
            wait_in(ca, n0)
            wait_out(ca - 2 * NUM_WORKERS, w0)
            widen(ca, n0, w0)
            fire_out(ca, w0)
            fire_in(cid(2 * i + 2), n0)
            wait_in(cb, n1)
            wait_out(cb - 2 * NUM_WORKERS, w1)
            widen(cb, n1, w1)
            fire_out(cb, w1)
            return carry

        lax.fori_loop(0, n_iter // 2, body, 0)
        wait_out(cid(n_iter - 2), w0)
        wait_out(cid(n_iter - 1), w1)

    return k1(table)


def _gather(fmt, idx, BATCH, HIST, D):
    batches_per_w = BATCH // NUM_WORKERS
    n_stages = batches_per_w // NB
    assert BATCH % NUM_WORKERS == 0 and batches_per_w % NB == 0
    assert n_stages % 2 == 0

    @pl.kernel(
        mesh=_make_mesh(),
        compiler_params=pltpu.CompilerParams(use_tc_tiling_on_sc=True),
        out_type=jax.ShapeDtypeStruct((BATCH, HIST, D), jnp.float32),
        scratch_types=[
            pltpu.VMEM((batches_per_w, HIST), jnp.int32),
            pltpu.VMEM((NB, HIST, 2 * D), jnp.float32),
            pltpu.VMEM((NB, HIST, 2 * D), jnp.float32),
            pltpu.VMEM((NB, HIST, D), jnp.float32),
            pltpu.SemaphoreType.DMA,
            pltpu.SemaphoreType.DMA,
        ],
    )
    def k2(fmt_hbm, idx_hbm, out_hbm, idx_v, buf0, buf1, obuf, sem0, sem1):
        wid = lax.axis_index("s") * NUM_CORES + lax.axis_index("c")
        base = wid * batches_per_w
        pltpu.sync_copy(idx_hbm.at[pl.ds(base, batches_per_w)], idx_v)

        def fire(st, buf, sem):
            for b in range(NB):
                pltpu.async_copy(
                    fmt_hbm.at[idx_v.at[st * NB + b]], buf.at[b], sem)

        def drain_out(st, buf, sem):
            for b in range(NB):
                pltpu.make_async_copy(
                    fmt_hbm.at[idx_v.at[st * NB + b]], buf.at[b], sem).wait()
            for b in range(NB):
                def row(h, carry):
                    for j in range(D // 16):
                        obuf[b, h, pl.ds(16 * j, 16)] = \
                            buf[b, h, pl.ds(16 * j, 16)]
                    return carry
                lax.fori_loop(0, HIST, row, 0)
            pltpu.sync_copy(obuf, out_hbm.at[pl.ds(base + st * NB, NB)])

        fire(0, buf0, sem0)

        def body(i, carry):
            s0 = 2 * i
            fire(s0 + 1, buf1, sem1)
            drain_out(s0, buf0, sem0)

            @pl.when(s0 + 2 < n_stages)
            def _():
                fire(s0 + 2, buf0, sem0)

            drain_out(s0 + 1, buf1, sem1)
            return carry

        lax.fori_loop(0, n_stages // 2, body, 0)

    return k2(fmt, idx)


def kernel(inputs, embedding):
    batch, hist = inputs.shape
    num_emb, feat = embedding.shape
    emb = jnp.asarray(embedding, jnp.float32)
    fmt = _format_table(emb, num_emb, feat)
    return _gather(fmt, inputs.astype(jnp.int32), batch, hist, feat)


# speedup vs baseline: 1.2344x; 1.2344x over previous
import jax
import jax.numpy as jnp
from jax import lax
from jax.experimental import pallas as pl
from jax.experimental.pallas import tpu as pltpu
from jax.experimental.pallas import tpu_sc as plsc

NUM_CORES = 2
NUM_SUBCORES = 16
NUM_WORKERS = NUM_CORES * NUM_SUBCORES

NB = 4


def _make_mesh():
    return plsc.VectorSubcoreMesh(
        core_axis_name="c", subcore_axis_name="s",
        num_cores=NUM_CORES, num_subcores=NUM_SUBCORES)


def _gather(fmt, idx, BATCH, HIST, D):
    batches_per_w = BATCH // NUM_WORKERS
    n_stages = batches_per_w // NB
    assert BATCH % NUM_WORKERS == 0 and batches_per_w % NB == 0
    assert n_stages % 2 == 0

    @pl.kernel(
        mesh=_make_mesh(),
        compiler_params=pltpu.CompilerParams(use_tc_tiling_on_sc=True),
        out_type=jax.ShapeDtypeStruct((BATCH, HIST, D), jnp.float32),
        scratch_types=[
            pltpu.VMEM((batches_per_w, HIST), jnp.int32),
            pltpu.VMEM((NB, HIST, 2 * D), jnp.float32),
            pltpu.VMEM((NB, HIST, 2 * D), jnp.float32),
            pltpu.VMEM((NB, HIST, D), jnp.float32),
            pltpu.SemaphoreType.DMA,
            pltpu.SemaphoreType.DMA,
        ],
    )
    def k2(fmt_hbm, idx_hbm, out_hbm, idx_v, buf0, buf1, obuf, sem0, sem1):
        wid = lax.axis_index("s") * NUM_CORES + lax.axis_index("c")
        base = wid * batches_per_w
        pltpu.sync_copy(idx_hbm.at[pl.ds(base, batches_per_w)], idx_v)

        def fire(st, buf, sem):
            for b in range(NB):
                pltpu.async_copy(
                    fmt_hbm.at[idx_v.at[st * NB + b]], buf.at[b], sem)

        def drain_out(st, buf, sem):
            for b in range(NB):
                pltpu.make_async_copy(
                    fmt_hbm.at[idx_v.at[st * NB + b]], buf.at[b], sem).wait()
            for b in range(NB):
                def row(h, carry):
                    for j in range(D // 16):
                        obuf[b, h, pl.ds(16 * j, 16)] = \
                            buf[b, h, pl.ds(16 * j, 16)]
                    return carry
                lax.fori_loop(0, HIST, row, 0)
            pltpu.sync_copy(obuf, out_hbm.at[pl.ds(base + st * NB, NB)])

        fire(0, buf0, sem0)

        def body(i, carry):
            s0 = 2 * i
            fire(s0 + 1, buf1, sem1)
            drain_out(s0, buf0, sem0)

            @pl.when(s0 + 2 < n_stages)
            def _():
                fire(s0 + 2, buf0, sem0)

            drain_out(s0 + 1, buf1, sem1)
            return carry

        lax.fori_loop(0, n_stages // 2, body, 0)

    return k2(fmt, idx)


def kernel(inputs, embedding):
    batch, hist = inputs.shape
    num_emb, feat = embedding.shape
    emb = jnp.asarray(embedding, jnp.float32)
    fmt = jnp.pad(emb, ((0, 0), (0, feat)))
    return _gather(fmt, inputs.astype(jnp.int32), batch, hist, feat)
